# baseline (speedup 1.0000x reference)
import functools

import jax
import jax.numpy as jnp
import numpy as np
from jax import lax
from jax.experimental import pallas as pl
from jax.experimental.pallas import tpu as pltpu
from jax.experimental.pallas import tpu_sc as plsc

HIDDEN = 768
QUARTER = HIDDEN // 4
BATCH = 16384

NHD = 24 * 7
NROWS = NHD + 52 + 12
NC, NS, L = 2, 16, 16
NW = NC * NS
BPW = BATCH // NW
G = 16
CHUNKS = BPW // G
HW = HIDDEN // 2

_EHD = np.zeros((NHD, 31), np.float32)
for _i in range(NHD):
    _EHD[_i, _i // 7] = 1.0
    _EHD[_i, 24 + _i % 7] = 1.0

_PERM = np.empty((HIDDEN,), np.int32)
for _j in range(HIDDEN // 32):
    for _i in range(L):
        _PERM[16 * _j + _i] = 32 * _j + _i
        _PERM[384 + 16 * _j + _i] = 32 * _j + 16 + _i
_P = np.zeros((HIDDEN, HIDDEN), np.float32)
for _p in range(HIDDEN):
    _P[_PERM[_p], _p] = 1.0


def _proj_body(th_ref, td_ref, tw_ref, tm_ref, w_ref, b_ref, ehd_ref,
               p_ref, o_ref):
    f32 = jnp.float32
    dn = (((1,), (1,)), ((), ()))
    ph = lax.dot_general(th_ref[...], w_ref[:, 0:QUARTER], dn,
                         preferred_element_type=f32)
    pd = lax.dot_general(td_ref[...], w_ref[:, QUARTER:2 * QUARTER], dn,
                         preferred_element_type=f32)
    pw = lax.dot_general(tw_ref[...], w_ref[:, 2 * QUARTER:3 * QUARTER], dn,
                         preferred_element_type=f32)
    pm = lax.dot_general(tm_ref[...], w_ref[:, 3 * QUARTER:], dn,
                         preferred_element_type=f32)
    hd = lax.dot_general(ehd_ref[...], jnp.concatenate([ph, pd], axis=0),
                         (((1,), (0,)), ((), ())),
                         preferred_element_type=f32) + b_ref[...]
    full = jnp.concatenate([hd, pw, pm], axis=0)
    full = lax.dot_general(full, p_ref[...], (((1,), (0,)), ((), ())),
                           preferred_element_type=f32)
    u16, u32 = jnp.uint16, jnp.uint32
    lo = lax.bitcast_convert_type(
        full[:, :HW].astype(jnp.bfloat16), u16).astype(u32)
    hi = lax.bitcast_convert_type(
        full[:, HW:].astype(jnp.bfloat16), u16).astype(u32)
    o_ref[...] = lax.bitcast_convert_type(lo | (hi << 16), f32)


def _sc_body(ptf_hbm, h_hbm, d_hbm, w_hbm, m_hbm, out_hbm,
             pt_v, hv, dv, wv, mv, o0, o1, sem_o):
    wid = lax.axis_index("s") * NC + lax.axis_index("c")
    base = wid * BPW
    pltpu.sync_copy(h_hbm.at[pl.ds(base, BPW)], hv)
    pltpu.sync_copy(d_hbm.at[pl.ds(base, BPW)], dv)
    pltpu.sync_copy(w_hbm.at[pl.ds(base, BPW)], wv)
    pltpu.sync_copy(m_hbm.at[pl.ds(base, BPW)], mv)
    pltpu.sync_copy(ptf_hbm, pt_v)

    @plsc.parallel_loop(0, BPW // L)
    def idx_body(j):
        off = j * L
        hv[pl.ds(off, L)] = (hv[pl.ds(off, L)] * 7 + dv[pl.ds(off, L)]) * HW
        wv[pl.ds(off, L)] = (wv[pl.ds(off, L)] + NHD) * HW
        mv[pl.ds(off, L)] = (mv[pl.ds(off, L)] + NHD + 52) * HW

    obufs = (o0, o1)

    def wait_out(phase):
        pltpu.make_async_copy(
            obufs[phase], out_hbm.at[pl.ds(0, G)], sem_o).wait()

    iota = lax.iota(jnp.int32, L)

    def pair_body(k, _):
        for phase in range(2):
            t = 2 * k + phase
            o = obufs[phase]
            hg = hv[pl.ds(t * G, L)]
            wg = wv[pl.ds(t * G, L)]
            mg = mv[pl.ds(t * G, L)]

            @pl.when(t >= 2)
            def _():
                wait_out(phase)

            @plsc.parallel_loop(0, G, unroll=8)
            def row_body(r):
                lv = jnp.broadcast_to(r, (L,))
                pib = "promise_in_bounds"
                ih = jnp.take_along_axis(hg, lv, axis=0, mode=pib) + iota
                iw = jnp.take_along_axis(wg, lv, axis=0, mode=pib) + iota
                im = jnp.take_along_axis(mg, lv, axis=0, mode=pib) + iota
                for c in range(HIDDEN // 32):
                    v1 = plsc.bitcast(plsc.load_gather(pt_v, [ih]),
                                      jnp.bfloat16)
                    v2 = plsc.bitcast(plsc.load_gather(pt_v, [iw]),
                                      jnp.bfloat16)
                    v3 = plsc.bitcast(plsc.load_gather(pt_v, [im]),
                                      jnp.bfloat16)
                    s2 = v2 + v3
                    l1, h1 = plsc.unpack(v1, format=plsc.PackFormat.INTERLEAVED)
                    l2, h2 = plsc.unpack(s2, format=plsc.PackFormat.INTERLEAVED)
                    o[r, pl.ds(c * 32, L)] = l1 + l2
                    o[r, pl.ds(c * 32 + L, L)] = h1 + h2
                    if c + 1 < HIDDEN // 32:
                        ih = ih + L
                        iw = iw + L
                        im = im + L

            pltpu.async_copy(o, out_hbm.at[pl.ds(base + t * G, G)], sem_o)
        return 0

    lax.fori_loop(0, CHUNKS // 2, pair_body, 0)
    wait_out(0)
    wait_out(1)


@jax.jit
def kernel(hours, days, weeks, months, hour_table, day_table, week_table,
           month_table, proj_w, proj_b):
    f32 = jnp.float32
    i32 = jnp.int32

    ptable = pl.pallas_call(
        _proj_body,
        out_shape=jax.ShapeDtypeStruct((NROWS, HW), f32),
    )(hour_table, day_table, week_table, month_table, proj_w,
      proj_b.reshape(1, HIDDEN), jnp.asarray(_EHD), jnp.asarray(_P))

    mesh = plsc.VectorSubcoreMesh(core_axis_name="c", subcore_axis_name="s")
    sc = functools.partial(
        pl.kernel,
        out_type=jax.ShapeDtypeStruct((BATCH, HIDDEN), f32),
        mesh=mesh,
        compiler_params=pltpu.CompilerParams(needs_layout_passes=False),
        scratch_types=[
            pltpu.VMEM((NROWS * HW,), f32),
            pltpu.VMEM((BPW,), i32),
            pltpu.VMEM((BPW,), i32),
            pltpu.VMEM((BPW,), i32),
            pltpu.VMEM((BPW,), i32),
            pltpu.VMEM((G, HIDDEN), f32),
            pltpu.VMEM((G, HIDDEN), f32),
            pltpu.SemaphoreType.DMA,
        ],
    )(_sc_body)
    return sc(ptable.reshape(NROWS * HW), hours.astype(i32),
              days.astype(i32), weeks.astype(i32), months.astype(i32))


# speedup vs baseline: 1.4666x; 1.4666x over previous
import functools

import jax
import jax.numpy as jnp
import numpy as np
from jax import lax
from jax.experimental import pallas as pl
from jax.experimental.pallas import tpu as pltpu
from jax.experimental.pallas import tpu_sc as plsc

HIDDEN = 768
QUARTER = HIDDEN // 4
BATCH = 16384

NHD = 24 * 7
NROWS = NHD + 52 + 12
NC, NS, L = 2, 16, 16
NW = NC * NS
BPW = BATCH // NW
G = 16
CHUNKS = BPW // G
HW = HIDDEN // 2

_EHD = np.zeros((NHD, 31), np.float32)
for _i in range(NHD):
    _EHD[_i, _i // 7] = 1.0
    _EHD[_i, 24 + _i % 7] = 1.0

_PERM = np.empty((HIDDEN,), np.int32)
for _j in range(HIDDEN // 32):
    for _i in range(L):
        _PERM[16 * _j + _i] = 32 * _j + _i
        _PERM[384 + 16 * _j + _i] = 32 * _j + 16 + _i
_P = np.zeros((HIDDEN, HIDDEN), np.float32)
for _p in range(HIDDEN):
    _P[_PERM[_p], _p] = 1.0


def _proj_body(th_ref, td_ref, tw_ref, tm_ref, w_ref, b_ref, ehd_ref,
               p_ref, o_ref):
    f32 = jnp.float32
    dn = (((1,), (1,)), ((), ()))
    ph = lax.dot_general(th_ref[...], w_ref[:, 0:QUARTER], dn,
                         preferred_element_type=f32)
    pd = lax.dot_general(td_ref[...], w_ref[:, QUARTER:2 * QUARTER], dn,
                         preferred_element_type=f32)
    pw = lax.dot_general(tw_ref[...], w_ref[:, 2 * QUARTER:3 * QUARTER], dn,
                         preferred_element_type=f32)
    pm = lax.dot_general(tm_ref[...], w_ref[:, 3 * QUARTER:], dn,
                         preferred_element_type=f32)
    hd = lax.dot_general(ehd_ref[...], jnp.concatenate([ph, pd], axis=0),
                         (((1,), (0,)), ((), ())),
                         preferred_element_type=f32) + b_ref[...]
    full = jnp.concatenate([hd, pw, pm], axis=0)
    full = lax.dot_general(full, p_ref[...], (((1,), (0,)), ((), ())),
                           preferred_element_type=f32)
    u16, u32 = jnp.uint16, jnp.uint32
    lo = lax.bitcast_convert_type(
        full[:, :HW].astype(jnp.bfloat16), u16).astype(u32)
    hi = lax.bitcast_convert_type(
        full[:, HW:].astype(jnp.bfloat16), u16).astype(u32)
    o_ref[...] = lax.bitcast_convert_type(lo | (hi << 16), f32)


def _sc_body(ptf_hbm, h_hbm, d_hbm, w_hbm, m_hbm, out_hbm,
             pt_v, hv, dv, wv, mv, o0, o1, sem_o, sem_p):
    wid = lax.axis_index("s") * NC + lax.axis_index("c")
    base = wid * BPW
    pltpu.async_copy(ptf_hbm, pt_v, sem_p)
    pltpu.async_copy(h_hbm.at[pl.ds(base, BPW)], hv, sem_o)
    pltpu.async_copy(d_hbm.at[pl.ds(base, BPW)], dv, sem_o)
    pltpu.async_copy(w_hbm.at[pl.ds(base, BPW)], wv, sem_o)
    pltpu.async_copy(m_hbm.at[pl.ds(base, BPW)], mv, sem_o)
    for buf, src in ((hv, h_hbm), (dv, d_hbm), (wv, w_hbm), (mv, m_hbm)):
        pltpu.make_async_copy(src.at[pl.ds(0, BPW)], buf, sem_o).wait()

    @plsc.parallel_loop(0, BPW // L)
    def idx_body(j):
        off = j * L
        hv[pl.ds(off, L)] = (hv[pl.ds(off, L)] * 7 + dv[pl.ds(off, L)]) * HW
        wv[pl.ds(off, L)] = (wv[pl.ds(off, L)] + NHD) * HW
        mv[pl.ds(off, L)] = (mv[pl.ds(off, L)] + NHD + 52) * HW

    pltpu.make_async_copy(ptf_hbm, pt_v, sem_p).wait()

    obufs = (o0, o1)

    def wait_out(phase):
        pltpu.make_async_copy(
            obufs[phase], out_hbm.at[pl.ds(0, G)], sem_o).wait()

    iota = lax.iota(jnp.int32, L)

    def pair_body(k, _):
        for phase in range(2):
            t = 2 * k + phase
            o = obufs[phase]
            hg = hv[pl.ds(t * G, L)]
            wg = wv[pl.ds(t * G, L)]
            mg = mv[pl.ds(t * G, L)]

            @pl.when(t >= 2)
            def _():
                wait_out(phase)

            @plsc.parallel_loop(0, G)
            def row_body(r):
                lv = jnp.broadcast_to(r, (L,))
                pib = "promise_in_bounds"
                ih = jnp.take_along_axis(hg, lv, axis=0, mode=pib) + iota
                iw = jnp.take_along_axis(wg, lv, axis=0, mode=pib) + iota
                im = jnp.take_along_axis(mg, lv, axis=0, mode=pib) + iota
                for c in range(HIDDEN // 32):
                    v1 = plsc.bitcast(plsc.load_gather(pt_v, [ih]),
                                      jnp.bfloat16)
                    v2 = plsc.bitcast(plsc.load_gather(pt_v, [iw]),
                                      jnp.bfloat16)
                    v3 = plsc.bitcast(plsc.load_gather(pt_v, [im]),
                                      jnp.bfloat16)
                    s2 = v2 + v3
                    l1, h1 = plsc.unpack(v1, format=plsc.PackFormat.INTERLEAVED)
                    l2, h2 = plsc.unpack(s2, format=plsc.PackFormat.INTERLEAVED)
                    o[r, pl.ds(c * 32, L)] = l1 + l2
                    o[r, pl.ds(c * 32 + L, L)] = h1 + h2
                    if c + 1 < HIDDEN // 32:
                        ih = ih + L
                        iw = iw + L
                        im = im + L

            pltpu.async_copy(o, out_hbm.at[pl.ds(base + t * G, G)], sem_o)
        return 0

    lax.fori_loop(0, CHUNKS // 2, pair_body, 0)
    wait_out(0)
    wait_out(1)


@jax.jit
def kernel(hours, days, weeks, months, hour_table, day_table, week_table,
           month_table, proj_w, proj_b):
    f32 = jnp.float32
    i32 = jnp.int32

    ptable = pl.pallas_call(
        _proj_body,
        out_shape=jax.ShapeDtypeStruct((NROWS, HW), f32),
    )(hour_table, day_table, week_table, month_table, proj_w,
      proj_b.reshape(1, HIDDEN), jnp.asarray(_EHD), jnp.asarray(_P))

    mesh = plsc.VectorSubcoreMesh(core_axis_name="c", subcore_axis_name="s")
    sc = functools.partial(
        pl.kernel,
        out_type=jax.ShapeDtypeStruct((BATCH, HIDDEN), f32),
        mesh=mesh,
        compiler_params=pltpu.CompilerParams(needs_layout_passes=False),
        scratch_types=[
            pltpu.VMEM((NROWS * HW,), f32),
            pltpu.VMEM((BPW,), i32),
            pltpu.VMEM((BPW,), i32),
            pltpu.VMEM((BPW,), i32),
            pltpu.VMEM((BPW,), i32),
            pltpu.VMEM((G, HIDDEN), f32),
            pltpu.VMEM((G, HIDDEN), f32),
            pltpu.SemaphoreType.DMA,
            pltpu.SemaphoreType.DMA,
        ],
    )(_sc_body)
    return sc(ptable.reshape(NROWS * HW), hours.astype(i32),
              days.astype(i32), weeks.astype(i32), months.astype(i32))
